# Initial kernel scaffold; baseline (speedup 1.0000x reference)
import jax
import jax.numpy as jnp
from jax.experimental import pallas as pl


def kernel(token_ids, weight):
    raise NotImplementedError("write your pallas kernel here")


# speedup vs baseline: 1.0944x; 1.0944x over previous
import functools

import jax
import jax.numpy as jnp
from jax import lax
from jax.experimental import pallas as pl
from jax.experimental.pallas import tpu as pltpu
from jax.experimental.pallas import tpu_sc as plsc

DIM = 32
NC = 2
NS = 16
NW = NC * NS

SUB = 128
SUBS_PER_CHUNK = 8
CHUNK = SUB * SUBS_PER_CHUNK


def _gather_body(idx_hbm, table_hbm, out_hbm, idx_v, rows_v, sem_g, *, b_per_w):
    n_chunks = b_per_w // CHUNK
    cid = lax.axis_index("c")
    sid = lax.axis_index("s")
    wid = sid * NC + cid
    base = wid * b_per_w

    def chunk_step(g, carry):
        row_off = base + g * CHUNK
        pltpu.sync_copy(
            idx_hbm.at[pl.ds(pl.multiple_of(row_off // SUB, 8), SUBS_PER_CHUNK)],
            idx_v,
        )
        copies = []
        for j in range(SUBS_PER_CHUNK):
            copies.append(
                pltpu.async_copy(
                    table_hbm.at[idx_v.at[j]],
                    rows_v.at[pl.ds(j * SUB, SUB)],
                    sem_g,
                )
            )
        for c in copies:
            c.wait()
        pltpu.sync_copy(rows_v, out_hbm.at[pl.ds(row_off, CHUNK)])
        return carry

    lax.fori_loop(0, n_chunks, chunk_step, 0)


@functools.partial(jax.jit, static_argnames=())
def kernel(token_ids, weight):
    b, t = token_ids.shape
    total = b * t
    b_per_w = total // NW
    idx_flat = token_ids.reshape(total).astype(jnp.int32)
    idx_2d = idx_flat.reshape(total // SUB, SUB)

    mesh = plsc.VectorSubcoreMesh(core_axis_name="c", subcore_axis_name="s")
    gathered = pl.kernel(
        functools.partial(_gather_body, b_per_w=b_per_w),
        out_type=jax.ShapeDtypeStruct((total, DIM), jnp.float32),
        mesh=mesh,
        scratch_types=[
            pltpu.VMEM((SUBS_PER_CHUNK, SUB), jnp.int32),
            pltpu.VMEM((CHUNK, DIM), jnp.float32),
            pltpu.SemaphoreType.DMA,
        ],
        compiler_params=pltpu.CompilerParams(use_tc_tiling_on_sc=False),
    )(idx_2d, weight)
    return gathered.reshape(b, t, DIM)


# speedup vs baseline: 1.2592x; 1.1506x over previous
import functools

import jax
import jax.numpy as jnp
from jax import lax
from jax.experimental import pallas as pl
from jax.experimental.pallas import tpu as pltpu
from jax.experimental.pallas import tpu_sc as plsc

NC = 2
NS = 16
NW = NC * NS

T = 50
B = 16384
D = 32
CB = 128
NCB = B // CB
TFULL = T // 8
TREM = T - 8 * TFULL
ITEMS_A = TFULL * NCB
ITEMS_B = NCB
PER_W_A = ITEMS_A // NW
PER_W_B = ITEMS_B // NW


def _splat16(v):
    return jnp.full((16,), v, jnp.int32)


_IOTA16 = None


def _body(tid_hbm, w_hbm, out_hbm, idx_v, gidx_v, qcol_v, grow_v, stage_v,
          sem_g):
    cid = lax.axis_index("c")
    sid = lax.axis_index("s")
    wid = sid * NC + cid
    iota16 = lax.iota(jnp.int32, 16)

    def prep_item(tbase, nrows, k):
        cb = k % NCB
        col0 = pl.multiple_of((k % NCB) * CB, 128)
        pltpu.sync_copy(
            tid_hbm.at[pl.ds(tbase, nrows), pl.ds(col0, CB)],
            idx_v.at[pl.ds(0, nrows)],
        )
        for rr in range(nrows):
            for jg in range(8):
                v = idx_v[rr, pl.ds(jg * 16, 16)]
                gidx_v[rr, pl.ds(jg * 16, 16)] = v >> 2
                qcol_v[rr, pl.ds(jg * 16, 16)] = (v & 3) * 32

    def sub_step(t_abs, r, k):
        cb = k % NCB
        col0 = pl.multiple_of(cb * CB, 128)
        rsplat = _splat16(r)
        pltpu.async_copy(w_hbm.at[gidx_v.at[r]], grow_v, sem_g).wait()
        for jg in range(8):
            rows = iota16 + (jg * 16)
            cols0 = plsc.load_gather(qcol_v, [rsplat, rows])
            for d in range(D):
                vals = plsc.load_gather(grow_v, [rows, cols0 + d])
                stage_v[d, pl.ds(jg * 16, 16)] = vals
        pltpu.sync_copy(stage_v, out_hbm.at[t_abs, :, pl.ds(col0, CB)])

    def step_a(s, carry):
        j = s // 8
        r = s % 8
        k = wid * PER_W_A + j

        @pl.when(r == 0)
        def _():
            prep_item(pl.multiple_of((k // NCB) * 8, 8), 8, k)

        sub_step((k // NCB) * 8 + r, r, k)
        return carry

    def step_b(s, carry):
        j = s // TREM
        r = s % TREM
        k = wid * PER_W_B + j

        @pl.when(r == 0)
        def _():
            prep_item(8 * TFULL, TREM, k)

        sub_step(8 * TFULL + r, r, k)
        return carry

    lax.fori_loop(0, PER_W_A * 8, step_a, 0)
    lax.fori_loop(0, PER_W_B * TREM, step_b, 0)


def kernel(token_ids, weight):
    tid_t = token_ids.T.astype(jnp.int32)
    w128 = weight.reshape(1000000 // 4, 128)
    mesh = plsc.VectorSubcoreMesh(core_axis_name="c", subcore_axis_name="s")
    ot = pl.kernel(
        _body,
        out_type=jax.ShapeDtypeStruct((T, D, B), jnp.float32),
        mesh=mesh,
        scratch_types=[
            pltpu.VMEM((8, CB), jnp.int32),
            pltpu.VMEM((8, CB), jnp.int32),
            pltpu.VMEM((8, CB), jnp.int32),
            pltpu.VMEM((CB, 128), jnp.float32),
            pltpu.VMEM((D, CB), jnp.float32),
            pltpu.SemaphoreType.DMA,
        ],
        compiler_params=pltpu.CompilerParams(needs_layout_passes=False),
    )(tid_t, w128)
    return jnp.transpose(ot, (2, 0, 1))


# speedup vs baseline: 1.6474x; 1.3083x over previous
import functools

import jax
import jax.numpy as jnp
from jax import lax
from jax.experimental import pallas as pl
from jax.experimental.pallas import tpu as pltpu
from jax.experimental.pallas import tpu_sc as plsc

NC = 2
NS = 16
NW = NC * NS

T = 50
B = 16384
D = 32
V = 1000000
CB = 128
NSTEP = T * B // (CB * NW)
OROWS = T * (D // 8) * (B // CB)


def _body(idx_hbm, w_hbm, o2_hbm, idxb_v, grow_v, stage_v, sem_i, sem_g,
          sem_o):
    cid = lax.axis_index("c")
    sid = lax.axis_index("s")
    wid = sid * NC + cid
    base = wid * NSTEP
    iota16 = lax.iota(jnp.int32, 16)
    rowvecs = [iota16 + (jg * 16) for jg in range(8)]

    def idx_slice(step):
        off = pl.multiple_of((base + step) * CB, 128)
        return idx_hbm.at[pl.ds(off, CB)]

    pltpu.sync_copy(idx_slice(0), idxb_v.at[0])
    pltpu.async_copy(idx_slice(1), idxb_v.at[1], sem_i)
    pltpu.async_copy(w_hbm.at[idxb_v.at[0]], grow_v.at[0], sem_g)

    def step_fn(s, carry):
        p = s % 2
        pp = 1 - p

        pltpu.make_async_copy(
            w_hbm.at[pl.ds(0, CB)], grow_v.at[pp], sem_g
        ).wait()

        @pl.when(s < NSTEP)
        def _():
            pltpu.make_async_copy(
                idx_slice(0), idxb_v.at[p], sem_i
            ).wait()

            @pl.when(s < NSTEP - 1)
            def _():
                pltpu.async_copy(idx_slice(s + 1), idxb_v.at[pp], sem_i)

            pltpu.async_copy(w_hbm.at[idxb_v.at[p]], grow_v.at[p], sem_g)

        @pl.when(s >= 3)
        def _():
            pltpu.make_async_copy(
                o2_hbm.at[pl.ds(0, 4)], stage_v.at[pp], sem_o
            ).wait()

        ppv = jnp.full((16,), pp, jnp.int32)
        for d in range(D):
            dv = jnp.full((16,), d, jnp.int32)
            R, r = d // 8, d % 8
            for jg in range(8):
                vals = plsc.load_gather(grow_v, [ppv, rowvecs[jg], dv])
                stage_v[pp, R, pl.ds(r * CB + jg * 16, 16)] = vals

        gr = base + s - 1
        t1 = gr // (B // CB)
        c1 = gr % (B // CB)
        rowbase = t1 * ((D // 8) * (B // CB)) + c1
        for R in range(4):
            pltpu.async_copy(
                stage_v.at[pp, R],
                o2_hbm.at[rowbase + R * (B // CB)],
                sem_o,
            )
        return carry

    lax.fori_loop(1, NSTEP + 1, step_fn, 0)

    for p in range(2):
        pltpu.make_async_copy(
            o2_hbm.at[pl.ds(0, 4)], stage_v.at[p], sem_o
        ).wait()


def kernel(token_ids, weight):
    idx_flat = token_ids.T.astype(jnp.int32).reshape(T * B)
    mesh = plsc.VectorSubcoreMesh(core_axis_name="c", subcore_axis_name="s")
    o2 = pl.kernel(
        _body,
        out_type=jax.ShapeDtypeStruct((OROWS, 1024), jnp.float32),
        mesh=mesh,
        scratch_types=[
            pltpu.VMEM((2, CB), jnp.int32),
            pltpu.VMEM((2, CB, D), jnp.float32),
            pltpu.VMEM((2, 4, 1024), jnp.float32),
            pltpu.SemaphoreType.DMA,
            pltpu.SemaphoreType.DMA,
            pltpu.SemaphoreType.DMA,
        ],
        compiler_params=pltpu.CompilerParams(
            use_tc_tiling_on_sc=False, needs_layout_passes=False
        ),
    )(idx_flat, weight)
    o5 = o2.reshape(T, D // 8, B // CB, 8, CB)
    return o5.transpose(2, 4, 0, 1, 3).reshape(B, T, D)


# speedup vs baseline: 2.3968x; 1.4549x over previous
import functools

import jax
import jax.numpy as jnp
from jax import lax
from jax.experimental import pallas as pl
from jax.experimental.pallas import tpu as pltpu
from jax.experimental.pallas import tpu_sc as plsc

NC = 2
NS = 16
NW = NC * NS

T = 50
B = 16384
D = 32
V = 1000000
CB = 128
JB = 4
NSTEP = T * JB
NCB = B // CB
OROWS = T * (D // 8) * NCB
GP = D + 1


def _body(idx_hbm, w_hbm, o2_hbm, idx_v, grow_v, stage_v, sem_i, sem_g,
          sem_o):
    cid = lax.axis_index("c")
    sid = lax.axis_index("s")
    wid = sid * NC + cid
    iota16 = lax.iota(jnp.int32, 16)
    rowvecs = [iota16 + (jg * 16) for jg in range(8)]

    for t in range(T):
        pltpu.async_copy(
            idx_hbm.at[pl.ds(t * NCB + wid * JB, JB)],
            idx_v.at[pl.ds(t * JB, JB)],
            sem_i,
        )
    pltpu.make_async_copy(
        idx_hbm.at[pl.ds(0, NSTEP)], idx_v, sem_i
    ).wait()

    def fire(s):
        pltpu.async_copy(
            w_hbm.at[idx_v.at[s]],
            grow_v.at[s % 4],
            sem_g,
        )

    for s in range(3):
        fire(s)

    def step_fn(s, carry):
        e = s - 3
        pltpu.make_async_copy(
            w_hbm.at[pl.ds(0, CB)],
            grow_v.at[0],
            sem_g,
        ).wait()

        @pl.when(s < NSTEP)
        def _():
            fire(s)

        ge = e % 4
        je = e % JB
        sr = (e // JB) % 2

        @pl.when(jnp.logical_and(je == 0, e >= 8))
        def _():
            for _R in range(D // 8):
                pltpu.make_async_copy(
                    o2_hbm.at[pl.ds(0, JB)],
                    stage_v.at[sr, :, pl.ds(0, 1024)],
                    sem_o,
                ).wait()

        gev = jnp.full((16,), ge, jnp.int32)
        srv = jnp.full((16,), sr, jnp.int32)
        jev = jnp.full((16,), je, jnp.int32)
        for d in range(D):
            colv = (iota16 + d) & (D - 1)
            colv128 = colv * CB
            for jg in range(8):
                vals = plsc.load_gather(grow_v, [gev, rowvecs[jg], colv])
                plsc.store_scatter(
                    stage_v, [srv, jev, colv128 + rowvecs[jg]], vals
                )

        @pl.when(je == JB - 1)
        def _():
            t1 = e // JB
            for R in range(D // 8):
                pltpu.async_copy(
                    stage_v.at[sr, :, pl.ds(R * 1024, 1024)],
                    o2_hbm.at[pl.ds(t1 * 512 + R * NCB + wid * JB, JB)],
                    sem_o,
                )
        return carry

    lax.fori_loop(3, NSTEP + 3, step_fn, 0)

    for _p in range(2):
        for _R in range(D // 8):
            pltpu.make_async_copy(
                o2_hbm.at[pl.ds(0, JB)],
                stage_v.at[0, :, pl.ds(0, 1024)],
                sem_o,
            ).wait()


def kernel(token_ids, weight):
    idx2d = token_ids.T.astype(jnp.int32).reshape(T * NCB, CB)
    mesh = plsc.VectorSubcoreMesh(core_axis_name="c", subcore_axis_name="s")
    o2 = pl.kernel(
        _body,
        out_type=jax.ShapeDtypeStruct((OROWS, 1024), jnp.float32),
        mesh=mesh,
        scratch_types=[
            pltpu.VMEM((NSTEP, CB), jnp.int32),
            pltpu.VMEM((4, CB, D), jnp.float32),
            pltpu.VMEM((2, JB, 4096), jnp.float32),
            pltpu.SemaphoreType.DMA,
            pltpu.SemaphoreType.DMA,
            pltpu.SemaphoreType.DMA,
        ],
        compiler_params=pltpu.CompilerParams(
            use_tc_tiling_on_sc=False, needs_layout_passes=False
        ),
    )(idx2d, weight)
    o5 = o2.reshape(T, D // 8, NCB, 8, CB)
    return o5.transpose(2, 4, 0, 1, 3).reshape(B, T, D)
